# baseline (speedup 1.0000x reference)
import functools

import jax
import jax.numpy as jnp
from jax import lax
from jax.experimental import pallas as pl
from jax.experimental.pallas import tpu as pltpu
from jax.experimental.pallas import tpu_sc as plsc

B = 16384
J = 26
N = B * J
D = 16
NW = 32
PER_W = N // NW
CHUNK = 1664
NCHUNK = PER_W // CHUNK


def _make_lookup():
    mesh = plsc.VectorSubcoreMesh(core_axis_name="c", subcore_axis_name="s")

    @functools.partial(
        pl.kernel,
        mesh=mesh,
        out_type=jax.ShapeDtypeStruct((N, D), jnp.float32),
        scratch_types=[
            pltpu.VMEM((CHUNK,), jnp.int32),
            pltpu.VMEM((CHUNK, D), jnp.float32),
            pltpu.SemaphoreType.DMA,
        ],
        compiler_params=pltpu.CompilerParams(use_tc_tiling_on_sc=False),
    )
    def body(idx_hbm, table_hbm, out_hbm, idx_v, rows_v, sem):
        wid = lax.axis_index("s") * 2 + lax.axis_index("c")
        base = wid * PER_W

        def step(c, _):
            off = base + c * CHUNK
            pltpu.sync_copy(idx_hbm.at[pl.ds(off, CHUNK)], idx_v)
            pltpu.async_copy(table_hbm.at[idx_v], rows_v, sem).wait()
            pltpu.sync_copy(rows_v, out_hbm.at[pl.ds(off, CHUNK)])
            return 0

        lax.fori_loop(0, NCHUNK, step, 0)

    return body


_lookup = _make_lookup()


@jax.jit
def kernel(node_ids, table):
    idx_flat = node_ids.reshape(N)
    out = _lookup(idx_flat, table)
    return out.reshape(B, J, D)


# speedup vs baseline: 1.0105x; 1.0105x over previous
import functools

import jax
import jax.numpy as jnp
from jax import lax
from jax.experimental import pallas as pl
from jax.experimental.pallas import tpu as pltpu
from jax.experimental.pallas import tpu_sc as plsc

B = 16384
J = 26
N = B * J
D = 16
NW = 32
PER_W = N // NW
CHUNK = 1664
NCHUNK = PER_W // CHUNK
NBUF = 4


def _make_lookup():
    mesh = plsc.VectorSubcoreMesh(core_axis_name="c", subcore_axis_name="s")

    scratch = [pltpu.VMEM((PER_W,), jnp.int32)]
    scratch += [pltpu.VMEM((CHUNK, D), jnp.float32) for _ in range(NBUF)]
    scratch += [pltpu.SemaphoreType.DMA for _ in range(2 * NBUF)]

    @functools.partial(
        pl.kernel,
        mesh=mesh,
        out_type=jax.ShapeDtypeStruct((N, D), jnp.float32),
        scratch_types=scratch,
        compiler_params=pltpu.CompilerParams(use_tc_tiling_on_sc=False),
    )
    def body(idx_hbm, table_hbm, out_hbm, idx_v, *bufs):
        rows = list(bufs[:NBUF])
        gsem = list(bufs[NBUF:2 * NBUF])
        osem = list(bufs[2 * NBUF:3 * NBUF])

        wid = lax.axis_index("s") * 2 + lax.axis_index("c")
        base = wid * PER_W

        pltpu.sync_copy(idx_hbm.at[pl.ds(base, PER_W)], idx_v)

        gather = [None] * NCHUNK
        out_cp = [None] * NCHUNK
        for c in range(min(NBUF, NCHUNK)):
            gather[c] = pltpu.async_copy(
                table_hbm.at[idx_v.at[pl.ds(c * CHUNK, CHUNK)]],
                rows[c % NBUF], gsem[c % NBUF])
        for c in range(NCHUNK):
            b = c % NBUF
            gather[c].wait()
            out_cp[c] = pltpu.async_copy(
                rows[b], out_hbm.at[pl.ds(base + c * CHUNK, CHUNK)], osem[b])
            nc = c + NBUF
            if nc < NCHUNK:
                out_cp[c].wait()
                gather[nc] = pltpu.async_copy(
                    table_hbm.at[idx_v.at[pl.ds(nc * CHUNK, CHUNK)]],
                    rows[b], gsem[b])
        for c in range(max(0, NCHUNK - NBUF), NCHUNK):
            out_cp[c].wait()

    return body


_lookup = _make_lookup()


@jax.jit
def kernel(node_ids, table):
    idx_flat = node_ids.reshape(N)
    out = _lookup(idx_flat, table)
    return out.reshape(B, J, D)


# speedup vs baseline: 4.9739x; 4.9224x over previous
import functools

import jax
import jax.numpy as jnp
from jax import lax
from jax.experimental import pallas as pl
from jax.experimental.pallas import tpu as pltpu
from jax.experimental.pallas import tpu_sc as plsc

B = 16384
J = 26
V = 1000000
D = 16
NS = 16
NC = 2
BPT = B // NS
DPC = D // NC
V_ALIGNED = 999936
TAIL = V - V_ALIGNED
STAGE = 62592
STAGE_LAST = V_ALIGNED - (NS - 1) * STAGE


def _make_lookup():
    mesh = plsc.VectorSubcoreMesh(core_axis_name="c", subcore_axis_name="s")

    @functools.partial(
        pl.kernel,
        mesh=mesh,
        out_type=jax.ShapeDtypeStruct((J, D, B), jnp.float32),
        scratch_types=[
            pltpu.VMEM_SHARED((V,), jnp.float32),
            pltpu.VMEM((J * BPT,), jnp.int32),
            pltpu.VMEM((J * BPT,), jnp.float32),
            pltpu.VMEM((D * 128,), jnp.float32),
            pltpu.SemaphoreType.DMA,
            pltpu.SemaphoreType.DMA,
            pltpu.SemaphoreType.DMA,
        ],
    )
    def body(idx_hbm, table_hbm, tail_hbm, out_hbm, row_sh, idx_v, vals_v,
             tail_v, gsem, ssem, osem):
        c = lax.axis_index("c")
        s = lax.axis_index("s")
        b0 = pl.multiple_of(s * BPT, 128)

        for j in range(J):
            pltpu.sync_copy(idx_hbm.at[j, pl.ds(b0, BPT)],
                            idx_v.at[pl.ds(j * BPT, BPT)])
        pltpu.sync_copy(tail_hbm, tail_v)

        for d in range(D):

            @pl.when(c == d // DPC)
            def _(d=d):
                @pl.when(s < NS - 1)
                def _():
                    off = pl.multiple_of(s * STAGE, 128)
                    pltpu.async_copy(
                        table_hbm.at[d, pl.ds(off, STAGE)],
                        row_sh.at[pl.ds(off, STAGE)], ssem).wait()

                @pl.when(s == NS - 1)
                def _():
                    off = (NS - 1) * STAGE
                    pltpu.async_copy(
                        table_hbm.at[d, pl.ds(off, STAGE_LAST)],
                        row_sh.at[pl.ds(off, STAGE_LAST)], ssem).wait()
                    pltpu.async_copy(
                        tail_v.at[pl.ds(d * 128, TAIL)],
                        row_sh.at[pl.ds(V_ALIGNED, TAIL)], ssem).wait()

                plsc.subcore_barrier()
                for j in range(J):
                    pltpu.async_copy(
                        row_sh.at[idx_v.at[pl.ds(j * BPT, BPT)]],
                        vals_v.at[pl.ds(j * BPT, BPT)], gsem).wait()
                    pltpu.async_copy(
                        vals_v.at[pl.ds(j * BPT, BPT)],
                        out_hbm.at[j, d, pl.ds(b0, BPT)], osem).wait()
                plsc.subcore_barrier()

    return body


_lookup = _make_lookup()


@jax.jit
def kernel(node_ids, table):
    tail = jnp.pad(table[V_ALIGNED:].T, ((0, 0), (0, 128 - TAIL))).reshape(-1)
    out_t = _lookup(node_ids.T, table.T, tail)
    return jnp.transpose(out_t, (2, 0, 1))


# speedup vs baseline: 6.3222x; 1.2711x over previous
import functools

import jax
import jax.numpy as jnp
from jax import lax
from jax.experimental import pallas as pl
from jax.experimental.pallas import tpu as pltpu
from jax.experimental.pallas import tpu_sc as plsc

B = 16384
J = 26
V = 1000000
D = 16
NS = 16
NC = 2
BPT = B // NS
DPC = D // NC
V_ALIGNED = 999936
TAIL = V - V_ALIGNED
STAGE = 62592
STAGE_LAST = V_ALIGNED - (NS - 1) * STAGE


def _make_lookup():
    mesh = plsc.VectorSubcoreMesh(core_axis_name="c", subcore_axis_name="s")

    @functools.partial(
        pl.kernel,
        mesh=mesh,
        out_type=jax.ShapeDtypeStruct((J, D, B), jnp.float32),
        scratch_types=[
            pltpu.VMEM_SHARED((V,), jnp.float32),
            pltpu.VMEM((J * BPT,), jnp.int32),
            pltpu.VMEM((J * BPT,), jnp.float32),
            pltpu.VMEM((DPC * 128,), jnp.float32),
            pltpu.SemaphoreType.DMA,
            pltpu.SemaphoreType.DMA,
            pltpu.SemaphoreType.DMA,
        ],
    )
    def body(idx_hbm, table_hbm, tail_hbm, out_hbm, row_sh, idx_v, vals_v,
             tail_v, gsem, ssem, osem):
        c = lax.axis_index("c")
        s = lax.axis_index("s")
        b0 = pl.multiple_of(s * BPT, 128)

        for j in range(J):
            pltpu.sync_copy(idx_hbm.at[j, pl.ds(b0, BPT)],
                            idx_v.at[pl.ds(j * BPT, BPT)])
        tb = pl.multiple_of(c * (DPC * 128), 128)
        pltpu.sync_copy(tail_hbm.at[pl.ds(tb, DPC * 128)], tail_v)

        def drain_writes():
            pltpu.make_async_copy(
                table_hbm.at[0, pl.ds(0, J * BPT)], vals_v, osem).wait()

        for d in range(D):

            @pl.when(c == d // DPC)
            def _(d=d):
                @pl.when(s < NS - 1)
                def _():
                    off = pl.multiple_of(s * STAGE, 128)
                    pltpu.async_copy(
                        table_hbm.at[d, pl.ds(off, STAGE)],
                        row_sh.at[pl.ds(off, STAGE)], ssem).wait()

                @pl.when(s == NS - 1)
                def _():
                    off = (NS - 1) * STAGE
                    pltpu.async_copy(
                        table_hbm.at[d, pl.ds(off, STAGE_LAST)],
                        row_sh.at[pl.ds(off, STAGE_LAST)], ssem).wait()
                    pltpu.async_copy(
                        tail_v.at[pl.ds((d % DPC) * 128, TAIL)],
                        row_sh.at[pl.ds(V_ALIGNED, TAIL)], ssem).wait()

                plsc.subcore_barrier()
                if d % DPC >= 1:
                    drain_writes()
                g_cps = [
                    pltpu.async_copy(
                        row_sh.at[idx_v.at[pl.ds(j * BPT, BPT)]],
                        vals_v.at[pl.ds(j * BPT, BPT)], gsem)
                    for j in range(J)
                ]
                for cp in g_cps:
                    cp.wait()
                plsc.subcore_barrier()
                for j in range(J):
                    pltpu.async_copy(
                        vals_v.at[pl.ds(j * BPT, BPT)],
                        out_hbm.at[j, d, pl.ds(b0, BPT)], osem)

        for cc in range(NC):

            @pl.when(c == cc)
            def _():
                drain_writes()

    return body


_lookup = _make_lookup()


@jax.jit
def kernel(node_ids, table):
    tail = jnp.pad(table[V_ALIGNED:].T, ((0, 0), (0, 128 - TAIL))).reshape(-1)
    out_t = _lookup(node_ids.T, table.T, tail)
    return jnp.transpose(out_t, (2, 0, 1))


# speedup vs baseline: 6.4660x; 1.0227x over previous
import functools

import jax
import jax.numpy as jnp
from jax import lax
from jax.experimental import pallas as pl
from jax.experimental.pallas import tpu as pltpu
from jax.experimental.pallas import tpu_sc as plsc

B = 16384
J = 26
V = 1000000
D = 16
NS = 16
NC = 2
BPT = B // NS
DPC = D // NC
V_ALIGNED = 999936
TAIL = V - V_ALIGNED
STAGE = 62592
STAGE_LAST = V_ALIGNED - (NS - 1) * STAGE


def _make_lookup():
    mesh = plsc.VectorSubcoreMesh(core_axis_name="c", subcore_axis_name="s")

    @functools.partial(
        pl.kernel,
        mesh=mesh,
        out_type=jax.ShapeDtypeStruct((J, D, B), jnp.float32),
        scratch_types=[
            pltpu.VMEM_SHARED((V,), jnp.float32),
            pltpu.VMEM((J * BPT,), jnp.int32),
            pltpu.VMEM((J * BPT,), jnp.float32),
            pltpu.VMEM((DPC * 128,), jnp.float32),
            pltpu.SemaphoreType.DMA,
            pltpu.SemaphoreType.DMA,
            pltpu.SemaphoreType.DMA,
        ],
    )
    def body(idx_hbm, table_hbm, tail_hbm, out_hbm, row_sh, idx_v, vals_v,
             tail_v, gsem, ssem, osem):
        c = lax.axis_index("c")
        s = lax.axis_index("s")
        b0 = pl.multiple_of(s * BPT, 128)

        for j in range(J):
            pltpu.sync_copy(idx_hbm.at[j, pl.ds(b0, BPT)],
                            idx_v.at[pl.ds(j * BPT, BPT)])
        tb = pl.multiple_of(c * (DPC * 128), 128)
        pltpu.sync_copy(tail_hbm.at[pl.ds(tb, DPC * 128)], tail_v)

        def drain_writes():
            pltpu.make_async_copy(
                table_hbm.at[0, pl.ds(0, J * BPT)], vals_v, osem).wait()

        for d in range(D):

            @pl.when(c == d // DPC)
            def _(d=d):
                @pl.when(s < NS - 1)
                def _():
                    off = pl.multiple_of(s * STAGE, 128)
                    pltpu.async_copy(
                        table_hbm.at[d, pl.ds(off, STAGE)],
                        row_sh.at[pl.ds(off, STAGE)], ssem).wait()

                @pl.when(s == NS - 1)
                def _():
                    off = (NS - 1) * STAGE
                    pltpu.async_copy(
                        table_hbm.at[d, pl.ds(off, STAGE_LAST)],
                        row_sh.at[pl.ds(off, STAGE_LAST)], ssem).wait()
                    pltpu.async_copy(
                        tail_v.at[pl.ds((d % DPC) * 128, TAIL)],
                        row_sh.at[pl.ds(V_ALIGNED, TAIL)], ssem).wait()

                plsc.subcore_barrier()
                if d % DPC >= 1:
                    drain_writes()
                pltpu.async_copy(row_sh.at[idx_v], vals_v, gsem).wait()
                plsc.subcore_barrier()
                for j in range(J):
                    pltpu.async_copy(
                        vals_v.at[pl.ds(j * BPT, BPT)],
                        out_hbm.at[j, d, pl.ds(b0, BPT)], osem)

        for cc in range(NC):

            @pl.when(c == cc)
            def _():
                drain_writes()

    return body


_lookup = _make_lookup()


@jax.jit
def kernel(node_ids, table):
    tail = jnp.pad(table[V_ALIGNED:].T, ((0, 0), (0, 128 - TAIL))).reshape(-1)
    out_t = _lookup(node_ids.T, table.T, tail)
    return jnp.transpose(out_t, (2, 0, 1))
